# Initial kernel scaffold; baseline (speedup 1.0000x reference)
import jax
import jax.numpy as jnp
from jax.experimental import pallas as pl


def kernel(x, noise, g1, g2, Wq, Wk, Wv, Wo, bo, Wg, bg, Wvar, bvar, W1, b1, W2, b2):
    raise NotImplementedError("write your pallas kernel here")


# speedup vs baseline: 1.0861x; 1.0861x over previous
import functools

import jax
import jax.numpy as jnp
from jax.experimental import pallas as pl
from jax.experimental.pallas import tpu as pltpu


TT = 256


def _qkv_kernel(x_ref, g_ref, w_ref, out_ref):
    xv = x_ref[...]
    h = xv * jax.lax.rsqrt(jnp.mean(xv * xv, axis=-1, keepdims=True) + 1e-6)
    h = h * g_ref[...]
    out_ref[...] = jnp.dot(h, w_ref[...], preferred_element_type=jnp.float32)


def _attn_kernel(q_ref, k_ref, v_ref, o_ref, *, scale, n_heads, dk):
    i = pl.program_id(0)
    q = q_ref[...]
    k = k_ref[...]
    v = v_ref[...]
    nq, nk = q.shape[0], k.shape[0]
    row = i * nq + jax.lax.broadcasted_iota(jnp.int32, (nq, nk), 0)
    col = jax.lax.broadcasted_iota(jnp.int32, (nq, nk), 1)
    causal = col <= row
    outs = []
    for h in range(n_heads):
        qh = q[:, h * dk:(h + 1) * dk]
        kh = k[:, h * dk:(h + 1) * dk]
        vh = v[:, h * dk:(h + 1) * dk]
        s = jax.lax.dot_general(qh, kh, (((1,), (1,)), ((), ())),
                                preferred_element_type=jnp.float32) * scale
        s = jnp.where(causal, s, -jnp.inf)
        m = jnp.max(s, axis=-1, keepdims=True)
        p = jnp.exp(s - m)
        p = p / jnp.sum(p, axis=-1, keepdims=True)
        outs.append(jnp.dot(p, vh, preferred_element_type=jnp.float32))
    o_ref[...] = jnp.concatenate(outs, axis=1)


def _proj_kernel(x_ref, o_ref, w_ref, b_ref, out_ref):
    out_ref[...] = (x_ref[...]
                    + jnp.dot(o_ref[...], w_ref[...],
                              preferred_element_type=jnp.float32)
                    + b_ref[...])


def _router_kernel(x_ref, g_ref, wg_ref, bg_ref, wv_ref, bv_ref, n_ref,
                   h2_ref, coef_ref):
    xv = x_ref[...]
    h2 = xv * jax.lax.rsqrt(jnp.mean(xv * xv, axis=-1, keepdims=True) + 1e-6)
    h2 = h2 * g_ref[...]
    h2_ref[...] = h2
    lg = jnp.dot(h2, wg_ref[...], preferred_element_type=jnp.float32) + bg_ref[...]
    lv = jnp.dot(h2, wv_ref[...], preferred_element_type=jnp.float32) + bv_ref[...]
    sp = jnp.maximum(lv, 0.0) + jnp.log(1.0 + jnp.exp(-jnp.abs(lv)))
    logits = lg + n_ref[...] * sp
    m1 = jnp.max(logits, axis=-1, keepdims=True)
    neg = jnp.where(logits == m1, -jnp.inf, logits)
    m2 = jnp.max(neg, axis=-1, keepdims=True)
    tmask = logits >= m2
    z = jnp.where(tmask, jnp.exp(logits - m1), 0.0)
    coef_ref[...] = z / jnp.sum(z, axis=-1, keepdims=True)


def _moe_kernel(x2_ref, h2_ref, coef_ref, w1_ref, b1_ref, w2_ref, b2_ref,
                out_ref, acc_ref, *, n_experts, tile):
    e = pl.program_id(0)
    i = pl.program_id(1)
    h2 = h2_ref[...]
    hm = jnp.maximum(
        jnp.dot(h2, w1_ref[0], preferred_element_type=jnp.float32)
        + b1_ref[0], 0.0)
    eiota = jax.lax.broadcasted_iota(jnp.int32, (1, n_experts), 1)
    c_col = jnp.sum(jnp.where(eiota == e, coef_ref[...], 0.0),
                    axis=-1, keepdims=True)
    contrib = (jnp.dot(c_col * hm, w2_ref[0],
                       preferred_element_type=jnp.float32)
               + c_col * b2_ref[0])

    @pl.when(e == 0)
    def _():
        acc_ref[pl.ds(i * tile, tile), :] = x2_ref[...] + contrib

    @pl.when(e > 0)
    def _():
        acc_ref[pl.ds(i * tile, tile), :] = (
            acc_ref[pl.ds(i * tile, tile), :] + contrib)

    @pl.when(e == n_experts - 1)
    def _():
        out_ref[...] = acc_ref[pl.ds(i * tile, tile), :]


def kernel(x, noise, g1, g2, Wq, Wk, Wv, Wo, bo, Wg, bg, Wvar, bvar,
           W1, b1, W2, b2):
    B, T, D = x.shape
    H, _, DK = Wq.shape
    E = Wg.shape[1]
    DFF = W1.shape[2]
    HD = H * DK
    tt = min(TT, T)
    nt = T // tt

    x2d = x.reshape(T, D)
    n2d = noise.reshape(T, E)
    g1r = g1.reshape(1, D)
    g2r = g2.reshape(1, D)
    bor = bo.reshape(1, D)
    bgr = bg.reshape(1, E)
    bvr = bvar.reshape(1, E)
    Wqkv = jnp.concatenate(
        [jnp.transpose(w, (1, 0, 2)).reshape(D, HD) for w in (Wq, Wk, Wv)],
        axis=1)

    qkv = pl.pallas_call(
        _qkv_kernel,
        grid=(nt,),
        in_specs=[
            pl.BlockSpec((tt, D), lambda i: (i, 0)),
            pl.BlockSpec((1, D), lambda i: (0, 0)),
            pl.BlockSpec((D, 3 * HD), lambda i: (0, 0)),
        ],
        out_specs=pl.BlockSpec((tt, 3 * HD), lambda i: (i, 0)),
        out_shape=jax.ShapeDtypeStruct((T, 3 * HD), jnp.float32),
    )(x2d, g1r, Wqkv)
    q2, k2, v2 = qkv[:, :HD], qkv[:, HD:2 * HD], qkv[:, 2 * HD:]

    o2 = pl.pallas_call(
        functools.partial(_attn_kernel, scale=1.0 / (DK ** 0.5),
                          n_heads=H, dk=DK),
        grid=(nt,),
        in_specs=[
            pl.BlockSpec((tt, HD), lambda i: (i, 0)),
            pl.BlockSpec((T, HD), lambda i: (0, 0)),
            pl.BlockSpec((T, HD), lambda i: (0, 0)),
        ],
        out_specs=pl.BlockSpec((tt, HD), lambda i: (i, 0)),
        out_shape=jax.ShapeDtypeStruct((T, HD), jnp.float32),
    )(q2, k2, v2)

    x2 = pl.pallas_call(
        _proj_kernel,
        grid=(nt,),
        in_specs=[
            pl.BlockSpec((tt, D), lambda i: (i, 0)),
            pl.BlockSpec((tt, D), lambda i: (i, 0)),
            pl.BlockSpec((D, D), lambda i: (0, 0)),
            pl.BlockSpec((1, D), lambda i: (0, 0)),
        ],
        out_specs=pl.BlockSpec((tt, D), lambda i: (i, 0)),
        out_shape=jax.ShapeDtypeStruct((T, D), jnp.float32),
    )(x2d, o2, Wo, bor)

    h2, coef = pl.pallas_call(
        _router_kernel,
        grid=(nt,),
        in_specs=[
            pl.BlockSpec((tt, D), lambda i: (i, 0)),
            pl.BlockSpec((1, D), lambda i: (0, 0)),
            pl.BlockSpec((D, E), lambda i: (0, 0)),
            pl.BlockSpec((1, E), lambda i: (0, 0)),
            pl.BlockSpec((D, E), lambda i: (0, 0)),
            pl.BlockSpec((1, E), lambda i: (0, 0)),
            pl.BlockSpec((tt, E), lambda i: (i, 0)),
        ],
        out_specs=[
            pl.BlockSpec((tt, D), lambda i: (i, 0)),
            pl.BlockSpec((tt, E), lambda i: (i, 0)),
        ],
        out_shape=[
            jax.ShapeDtypeStruct((T, D), jnp.float32),
            jax.ShapeDtypeStruct((T, E), jnp.float32),
        ],
    )(x2, g2r, Wg, bgr, Wvar, bvr, n2d)

    out = pl.pallas_call(
        functools.partial(_moe_kernel, n_experts=E, tile=tt),
        grid=(E, nt),
        in_specs=[
            pl.BlockSpec((tt, D), lambda e, i: (i, 0)),
            pl.BlockSpec((tt, D), lambda e, i: (i, 0)),
            pl.BlockSpec((tt, E), lambda e, i: (i, 0)),
            pl.BlockSpec((1, D, DFF), lambda e, i: (e, 0, 0)),
            pl.BlockSpec((1, 1, DFF), lambda e, i: (e, 0, 0)),
            pl.BlockSpec((1, DFF, D), lambda e, i: (e, 0, 0)),
            pl.BlockSpec((1, 1, D), lambda e, i: (e, 0, 0)),
        ],
        out_specs=pl.BlockSpec((tt, D), lambda e, i: (i, 0)),
        out_shape=jax.ShapeDtypeStruct((T, D), jnp.float32),
        scratch_shapes=[pltpu.VMEM((T, D), jnp.float32)],
    )(x2, h2, coef, W1, b1.reshape(E, 1, DFF), W2, b2.reshape(E, 1, D))

    return out.reshape(B, T, D)
